# baseline (speedup 1.0000x reference)
import functools

import jax
import jax.numpy as jnp
from jax import lax
from jax.experimental import pallas as pl
from jax.experimental.pallas import tpu as pltpu
from jax.experimental.pallas import tpu_sc as plsc

N = 10000
E = 320000
D_IN = 128
F = 64
OUT = 10
G = 64

NC = 2
NS = 16
NP = 10240
RPT = NP // NS
EPT = E // NS
CHUNK = 80
NCHUNK = EPT // CHUNK
NQ = NCHUNK // 4


def _tc_prep1_body(x_ref, w_ref, as_ref, ad_ref, h_ref, asr_ref, adr_ref, m_ref):
    h = jnp.dot(x_ref[...], w_ref[...], preferred_element_type=jnp.float32)
    h_ref[0] = h[:, :F // 2]
    h_ref[1] = h[:, F // 2:]
    asr = jnp.sum(h * as_ref[...], axis=1, keepdims=True)
    adr = jnp.sum(h * ad_ref[...], axis=1, keepdims=True)
    asr_ref[...] = asr
    adr_ref[...] = adr
    m = jnp.maximum(jnp.max(asr) + jnp.max(adr), 0.0)
    m_ref[...] = jnp.full((8, 128), m, jnp.float32)


def _tc_prep2_body(o_ref, b_ref, w_ref, as_ref, ad_ref,
                   xl_ref, h_ref, asr_ref, adr_ref, m_ref):
    o = jnp.concatenate([o_ref[0, :N, :], o_ref[1, :N, :]], axis=1)
    xl = jnp.maximum(o + b_ref[...], 0.0)
    xl_ref[...] = xl
    h = jnp.dot(xl, w_ref[...], preferred_element_type=jnp.float32)
    h_ref[0] = h[:, :F // 2]
    h_ref[1] = h[:, F // 2:]
    asr = jnp.sum(h * as_ref[...], axis=1, keepdims=True)
    adr = jnp.sum(h * ad_ref[...], axis=1, keepdims=True)
    asr_ref[...] = asr
    adr_ref[...] = adr
    m = jnp.maximum(jnp.max(asr) + jnp.max(adr), 0.0)
    m_ref[...] = jnp.full((8, 128), m, jnp.float32)


def _tc_final_body(x1_ref, x2_ref, o_ref, b3_ref, batch_ref,
                   wl_ref, bl_ref, out_ref):
    o = jnp.concatenate([o_ref[0, :N, :], o_ref[1, :N, :]], axis=1)
    x3 = jnp.maximum(o + b3_ref[...], 0.0)
    xc = jnp.concatenate([x1_ref[...], x2_ref[...], x3], axis=1)
    gid = lax.broadcasted_iota(jnp.int32, (N, G), 1)
    oh = (batch_ref[...] == gid).astype(jnp.float32)
    sums = lax.dot_general(oh, xc, (((0,), (0,)), ((), ())),
                           preferred_element_type=jnp.float32)
    ones = jnp.ones((N, 1), jnp.float32)
    counts = lax.dot_general(oh, ones, (((0,), (0,)), ((), ())),
                             preferred_element_type=jnp.float32)
    pooled = sums / jnp.maximum(counts, 1.0)
    logits = jnp.dot(pooled, wl_ref[...],
                     preferred_element_type=jnp.float32) + bl_ref[...]
    z = logits - jnp.max(logits, axis=1, keepdims=True)
    ez = jnp.exp(z)
    out_ref[...] = ez / jnp.sum(ez, axis=1, keepdims=True)


def _tc_prep1(x, w, a_s, a_d):
    return pl.pallas_call(
        _tc_prep1_body,
        out_shape=[
            jax.ShapeDtypeStruct((NC, N, F // 2), jnp.float32),
            jax.ShapeDtypeStruct((N, 1), jnp.float32),
            jax.ShapeDtypeStruct((N, 1), jnp.float32),
            jax.ShapeDtypeStruct((8, 128), jnp.float32),
        ],
    )(x, w, a_s.reshape(1, F), a_d.reshape(1, F))


def _tc_prep2(o, b, w, a_s, a_d):
    return pl.pallas_call(
        _tc_prep2_body,
        out_shape=[
            jax.ShapeDtypeStruct((N, F), jnp.float32),
            jax.ShapeDtypeStruct((NC, N, F // 2), jnp.float32),
            jax.ShapeDtypeStruct((N, 1), jnp.float32),
            jax.ShapeDtypeStruct((N, 1), jnp.float32),
            jax.ShapeDtypeStruct((8, 128), jnp.float32),
        ],
    )(o, b.reshape(1, F), w, a_s.reshape(1, F), a_d.reshape(1, F))


def _tc_final(x1, x2, o, b3, batch, wl, bl):
    return pl.pallas_call(
        _tc_final_body,
        out_shape=jax.ShapeDtypeStruct((G, OUT), jnp.float32),
    )(x1, x2, o, b3.reshape(1, F), batch.reshape(N, 1), wl,
      bl.reshape(1, OUT))


_SC_MESH = plsc.VectorSubcoreMesh(core_axis_name="c", subcore_axis_name="s")


@functools.partial(
    pl.kernel,
    out_type=jax.ShapeDtypeStruct((NC, NP, F // 2), jnp.float32),
    mesh=_SC_MESH,
    compiler_params=pltpu.CompilerParams(
        needs_layout_passes=False, use_tc_tiling_on_sc=False),
    scratch_types=[
        pltpu.VMEM((NCHUNK, CHUNK), jnp.int32),
        pltpu.VMEM((NCHUNK, CHUNK), jnp.int32),
        pltpu.VMEM((NP,), jnp.float32),
        pltpu.VMEM((NP // CHUNK, CHUNK), jnp.int32),
        pltpu.VMEM((N,), jnp.float32),
        pltpu.VMEM((N,), jnp.float32),
        pltpu.VMEM((CHUNK, F // 2), jnp.float32),
        pltpu.VMEM((CHUNK, F // 2), jnp.float32),
        pltpu.VMEM((CHUNK, F // 2), jnp.float32),
        pltpu.VMEM((CHUNK, F // 2), jnp.float32),
        pltpu.VMEM((RPT,), jnp.float32),
        pltpu.VMEM((16,), jnp.float32),
        pltpu.VMEM_SHARED((NP,), jnp.float32),
        pltpu.VMEM_SHARED((NP, F // 2), jnp.float32),
        pltpu.SemaphoreType.DMA,
        pltpu.SemaphoreType.DMA,
        pltpu.SemaphoreType.DMA,
        pltpu.SemaphoreType.DMA,
        pltpu.SemaphoreType.DMA,
        pltpu.SemaphoreType.DMA,
        pltpu.SemaphoreType.DMA,
        pltpu.SemaphoreType.DMA,
        pltpu.SemaphoreType.DMA,
    ],
)
def _sc_layer(h_hbm, as_hbm, ad_hbm, m_hbm, src_hbm, dst_hbm, out_hbm,
              src_v, dst_v, denloc, idx_id, as_v, ad_v, rb0, rb1, rb2, rb3,
              dbuf, m_v, den_sh, out_sh, semd, g0, g1, g2, g3, s0, s1, s2, s3):
    sid = lax.axis_index("s")
    cid = lax.axis_index("c")
    bufs = (rb0, rb1, rb2, rb3)
    gsems = (g0, g1, g2, g3)
    ssems = (s0, s1, s2, s3)

    pltpu.sync_copy(src_hbm.at[sid], src_v)
    pltpu.sync_copy(dst_hbm.at[sid], dst_v)
    pltpu.sync_copy(as_hbm, as_v)
    pltpu.sync_copy(ad_hbm, ad_v)
    pltpu.sync_copy(m_hbm.at[0, pl.ds(0, 16)], m_v)

    zero16 = jnp.zeros((16,), jnp.float32)
    zero16i = jnp.zeros((16,), jnp.int32)

    def _zrow(r, carry):
        for c in range(F // 32):
            rb0[r, pl.ds(c * 16, 16)] = zero16
        return carry
    lax.fori_loop(0, CHUNK, _zrow, 0)
    for k in range(RPT // CHUNK):
        pltpu.sync_copy(rb0, out_sh.at[pl.ds(sid * RPT + k * CHUNK, CHUNK)])

    def _zden(r, carry):
        dbuf[pl.ds(r * 16, 16)] = zero16
        return carry
    lax.fori_loop(0, RPT // 16, _zden, 0)
    pltpu.sync_copy(dbuf, den_sh.at[pl.ds(sid * RPT, RPT)])

    iota16 = lax.iota(jnp.int32, 16)

    def _zdl(q, carry):
        for g in range(CHUNK // 16):
            denloc[pl.ds(q * CHUNK + g * 16, 16)] = zero16
            idx_id[q, pl.ds(g * 16, 16)] = iota16 + (q * CHUNK + g * 16)
        return carry
    lax.fori_loop(0, NP // CHUNK, _zdl, 0)

    plsc.subcore_barrier()

    m_vec = m_v[...]
    h_half = h_hbm.at[cid]

    def _gat(j, b):
        return pltpu.make_async_copy(h_half.at[src_v.at[j]], bufs[b], gsems[b])

    def _sct(j, b):
        return pltpu.make_async_copy(bufs[b], out_sh.at[dst_v.at[j]], ssems[b])

    def _proc(j, b):
        buf = bufs[b]
        for g in range(CHUNK // 16):
            sv = src_v[j, pl.ds(g * 16, 16)]
            dv = dst_v[j, pl.ds(g * 16, 16)]
            e = plsc.load_gather(as_v, [sv]) + plsc.load_gather(ad_v, [dv])
            e = jnp.where(e >= 0.0, e, e * 0.2)
            w = jnp.exp(e - m_vec)
            plsc.addupdate_scatter(denloc, [dv], w)
            for i in range(16):
                a = w[i]
                r = g * 16 + i
                for c in range(F // 32):
                    buf[r, pl.ds(c * 16, 16)] = buf[r, pl.ds(c * 16, 16)] * a
        _sct(j, b).start(add=True)

    def _step(j, b):
        _gat(j, b).wait()
        _proc(j, b)
        b2 = (b + 2) % 4

        @pl.when(j >= 2)
        def _():
            _sct(j - 2, b2).wait()

        @pl.when(j + 2 < NCHUNK)
        def _():
            _gat(j + 2, b2).start()

    scope = jax.named_scope("fused_edge_pass")
    scope.__enter__()
    _gat(0, 0).start()
    _gat(1, 1).start()

    def _ring(t, carry):
        j0 = 4 * t
        for b in range(4):
            _step(j0 + b, b)
        return carry
    lax.fori_loop(0, NQ, _ring, 0)
    _step(jnp.int32(NCHUNK - 2), 0)
    _step(jnp.int32(NCHUNK - 1), 1)
    _sct(NCHUNK - 2, 0).wait()
    _sct(NCHUNK - 1, 1).wait()

    def _dpush(q, carry):
        pltpu.async_copy(denloc.at[pl.ds(q * CHUNK, CHUNK)],
                         den_sh.at[idx_id.at[q]], semd, add=True)
        return carry
    lax.fori_loop(0, NP // CHUNK, _dpush, 0)

    def _dwait(q, carry):
        pltpu.make_async_copy(denloc.at[pl.ds(q * CHUNK, CHUNK)],
                              den_sh.at[idx_id.at[q]], semd).wait()
        return carry
    lax.fori_loop(0, NP // CHUNK, _dwait, 0)
    scope.__exit__(None, None, None)

    plsc.subcore_barrier()

    pltpu.sync_copy(den_sh.at[pl.ds(sid * RPT, RPT)], dbuf)
    one16 = jnp.ones((16,), jnp.float32)

    for k in range(RPT // CHUNK):
        base = sid * RPT + k * CHUNK
        pltpu.sync_copy(out_sh.at[pl.ds(base, CHUNK)], rb0)
        for g in range(CHUNK // 16):
            den = dbuf[pl.ds(k * CHUNK + g * 16, 16)]
            rden = one16 / (den + 1e-16)
            for i in range(16):
                a = rden[i]
                r = g * 16 + i
                for c in range(F // 32):
                    rb0[r, pl.ds(c * 16, 16)] = rb0[r, pl.ds(c * 16, 16)] * a
        pltpu.sync_copy(rb0, out_hbm.at[cid, pl.ds(base, CHUNK)])


def kernel(x, edge_index, batch, W1, a1s, a1d, b1, W2, a2s, a2d, b2,
           W3, a3s, a3d, b3, Wl, bl):
    src3 = edge_index[0].reshape(NS, NCHUNK, CHUNK)
    dst3 = edge_index[1].reshape(NS, NCHUNK, CHUNK)

    h1, as1, ad1, m1 = _tc_prep1(x, W1, a1s, a1d)
    o1 = _sc_layer(h1, as1.reshape(N), ad1.reshape(N), m1, src3, dst3)
    x1, h2, as2, ad2, m2 = _tc_prep2(o1, b1, W2, a2s, a2d)
    o2 = _sc_layer(h2, as2.reshape(N), ad2.reshape(N), m2, src3, dst3)
    x2, h3, as3, ad3, m3 = _tc_prep2(o2, b2, W3, a3s, a3d)
    o3 = _sc_layer(h3, as3.reshape(N), ad3.reshape(N), m3, src3, dst3)
    return _tc_final(x1, x2, o3, b3, batch, Wl, bl)


# speedup vs baseline: 1.0789x; 1.0789x over previous
import functools

import jax
import jax.numpy as jnp
from jax import lax
from jax.experimental import pallas as pl
from jax.experimental.pallas import tpu as pltpu
from jax.experimental.pallas import tpu_sc as plsc

N = 10000
E = 320000
D_IN = 128
F = 64
OUT = 10
G = 64

NC = 2
NS = 16
NP = 10240
RPT = NP // NS
EPT = E // NS
CHUNK = 80
NCHUNK = EPT // CHUNK
NQ = NCHUNK // 4


def _attention_rows(h, as_ref, ad_ref):
    asr = lax.dot_general(as_ref[...], h, (((1,), (1,)), ((), ())),
                          preferred_element_type=jnp.float32)
    adr = lax.dot_general(ad_ref[...], h, (((1,), (1,)), ((), ())),
                          preferred_element_type=jnp.float32)
    asad = jnp.concatenate([asr, adr], axis=0)
    m = jnp.maximum(jnp.max(asr) + jnp.max(adr), 0.0)
    return asad, m


def _tc_prep1_body(x_ref, w_ref, as_ref, ad_ref, h_ref, asad_ref, m_ref):
    h = jnp.dot(x_ref[...], w_ref[...], preferred_element_type=jnp.float32)
    h_ref[0] = h[:, :F // 2]
    h_ref[1] = h[:, F // 2:]
    asad, m = _attention_rows(h, as_ref, ad_ref)
    asad_ref[...] = asad
    m_ref[...] = jnp.full((8, 128), m, jnp.float32)


def _tc_prep2_body(o_ref, b_ref, w_ref, as_ref, ad_ref,
                   xl_ref, h_ref, asad_ref, m_ref):
    o = jnp.concatenate([o_ref[0, :N, :], o_ref[1, :N, :]], axis=1)
    xl = jnp.maximum(o + b_ref[...], 0.0)
    xl_ref[...] = xl
    h = jnp.dot(xl, w_ref[...], preferred_element_type=jnp.float32)
    h_ref[0] = h[:, :F // 2]
    h_ref[1] = h[:, F // 2:]
    asad, m = _attention_rows(h, as_ref, ad_ref)
    asad_ref[...] = asad
    m_ref[...] = jnp.full((8, 128), m, jnp.float32)


def _tc_final_body(x1_ref, x2_ref, o_ref, b3_ref, batch_ref,
                   wl_ref, bl_ref, out_ref):
    o = jnp.concatenate([o_ref[0, :N, :], o_ref[1, :N, :]], axis=1)
    x3 = jnp.maximum(o + b3_ref[...], 0.0)
    xc = jnp.concatenate([x1_ref[...], x2_ref[...], x3], axis=1)
    gid = lax.broadcasted_iota(jnp.int32, (G, N), 0)
    oh_t = (batch_ref[...] == gid).astype(jnp.float32)
    sums = lax.dot_general(oh_t, xc, (((1,), (0,)), ((), ())),
                           preferred_element_type=jnp.float32)
    ones = jnp.ones((N, 1), jnp.float32)
    counts = lax.dot_general(oh_t, ones, (((1,), (0,)), ((), ())),
                             preferred_element_type=jnp.float32)
    pooled = sums / jnp.maximum(counts, 1.0)
    logits = jnp.dot(pooled, wl_ref[...],
                     preferred_element_type=jnp.float32) + bl_ref[...]
    z = logits - jnp.max(logits, axis=1, keepdims=True)
    ez = jnp.exp(z)
    out_ref[...] = ez / jnp.sum(ez, axis=1, keepdims=True)


def _tc_prep1(x, w, a_s, a_d):
    return pl.pallas_call(
        _tc_prep1_body,
        out_shape=[
            jax.ShapeDtypeStruct((NC, N, F // 2), jnp.float32),
            jax.ShapeDtypeStruct((2, N), jnp.float32),
            jax.ShapeDtypeStruct((8, 128), jnp.float32),
        ],
    )(x, w, a_s.reshape(1, F), a_d.reshape(1, F))


def _tc_prep2(o, b, w, a_s, a_d):
    return pl.pallas_call(
        _tc_prep2_body,
        out_shape=[
            jax.ShapeDtypeStruct((N, F), jnp.float32),
            jax.ShapeDtypeStruct((NC, N, F // 2), jnp.float32),
            jax.ShapeDtypeStruct((2, N), jnp.float32),
            jax.ShapeDtypeStruct((8, 128), jnp.float32),
        ],
    )(o, b.reshape(1, F), w, a_s.reshape(1, F), a_d.reshape(1, F))


def _tc_final(x1, x2, o, b3, batch, wl, bl):
    return pl.pallas_call(
        _tc_final_body,
        out_shape=jax.ShapeDtypeStruct((G, OUT), jnp.float32),
    )(x1, x2, o, b3.reshape(1, F), batch.reshape(1, N), wl,
      bl.reshape(1, OUT))


_SC_MESH = plsc.VectorSubcoreMesh(core_axis_name="c", subcore_axis_name="s")


@functools.partial(
    pl.kernel,
    out_type=jax.ShapeDtypeStruct((NC, NP, F // 2), jnp.float32),
    mesh=_SC_MESH,
    compiler_params=pltpu.CompilerParams(
        needs_layout_passes=False, use_tc_tiling_on_sc=False),
    scratch_types=[
        pltpu.VMEM((NCHUNK, CHUNK), jnp.int32),
        pltpu.VMEM((NCHUNK, CHUNK), jnp.int32),
        pltpu.VMEM((NP,), jnp.float32),
        pltpu.VMEM((NP // CHUNK, CHUNK), jnp.int32),
        pltpu.VMEM((N,), jnp.float32),
        pltpu.VMEM((N,), jnp.float32),
        pltpu.VMEM((CHUNK, F // 2), jnp.float32),
        pltpu.VMEM((CHUNK, F // 2), jnp.float32),
        pltpu.VMEM((CHUNK, F // 2), jnp.float32),
        pltpu.VMEM((CHUNK, F // 2), jnp.float32),
        pltpu.VMEM((RPT,), jnp.float32),
        pltpu.VMEM((16,), jnp.float32),
        pltpu.VMEM_SHARED((NP,), jnp.float32),
        pltpu.VMEM_SHARED((NP, F // 2), jnp.float32),
        pltpu.SemaphoreType.DMA,
        pltpu.SemaphoreType.DMA,
        pltpu.SemaphoreType.DMA,
        pltpu.SemaphoreType.DMA,
        pltpu.SemaphoreType.DMA,
        pltpu.SemaphoreType.DMA,
        pltpu.SemaphoreType.DMA,
        pltpu.SemaphoreType.DMA,
        pltpu.SemaphoreType.DMA,
    ],
)
def _sc_layer(h_hbm, asad_hbm, m_hbm, src_hbm, dst_hbm, out_hbm,
              src_v, dst_v, denloc, idx_id, as_v, ad_v, rb0, rb1, rb2, rb3,
              dbuf, m_v, den_sh, out_sh, semd, g0, g1, g2, g3, s0, s1, s2, s3):
    sid = lax.axis_index("s")
    cid = lax.axis_index("c")
    bufs = (rb0, rb1, rb2, rb3)
    gsems = (g0, g1, g2, g3)
    ssems = (s0, s1, s2, s3)

    pltpu.sync_copy(src_hbm.at[sid], src_v)
    pltpu.sync_copy(dst_hbm.at[sid], dst_v)
    pltpu.sync_copy(asad_hbm.at[0], as_v)
    pltpu.sync_copy(asad_hbm.at[1], ad_v)
    pltpu.sync_copy(m_hbm.at[0, pl.ds(0, 16)], m_v)

    zero16 = jnp.zeros((16,), jnp.float32)
    zero16i = jnp.zeros((16,), jnp.int32)

    def _zrow(r, carry):
        for c in range(F // 32):
            rb0[r, pl.ds(c * 16, 16)] = zero16
        return carry
    lax.fori_loop(0, CHUNK, _zrow, 0)
    for k in range(RPT // CHUNK):
        pltpu.sync_copy(rb0, out_sh.at[pl.ds(sid * RPT + k * CHUNK, CHUNK)])

    def _zden(r, carry):
        dbuf[pl.ds(r * 16, 16)] = zero16
        return carry
    lax.fori_loop(0, RPT // 16, _zden, 0)
    pltpu.sync_copy(dbuf, den_sh.at[pl.ds(sid * RPT, RPT)])

    iota16 = lax.iota(jnp.int32, 16)

    def _zdl(q, carry):
        for g in range(CHUNK // 16):
            denloc[pl.ds(q * CHUNK + g * 16, 16)] = zero16
            idx_id[q, pl.ds(g * 16, 16)] = iota16 + (q * CHUNK + g * 16)
        return carry
    lax.fori_loop(0, NP // CHUNK, _zdl, 0)

    plsc.subcore_barrier()

    m_vec = m_v[...]
    h_half = h_hbm.at[cid]

    def _gat(j, b):
        return pltpu.make_async_copy(h_half.at[src_v.at[j]], bufs[b], gsems[b])

    def _sct(j, b):
        return pltpu.make_async_copy(bufs[b], out_sh.at[dst_v.at[j]], ssems[b])

    def _proc(j, b):
        buf = bufs[b]
        for g in range(CHUNK // 16):
            sv = src_v[j, pl.ds(g * 16, 16)]
            dv = dst_v[j, pl.ds(g * 16, 16)]
            e = plsc.load_gather(as_v, [sv]) + plsc.load_gather(ad_v, [dv])
            e = jnp.where(e >= 0.0, e, e * 0.2)
            w = jnp.exp(e - m_vec)
            plsc.addupdate_scatter(denloc, [dv], w)
            for i in range(16):
                a = w[i]
                r = g * 16 + i
                for c in range(F // 32):
                    buf[r, pl.ds(c * 16, 16)] = buf[r, pl.ds(c * 16, 16)] * a
        _sct(j, b).start(add=True)

    def _step(j, b):
        _gat(j, b).wait()
        _proc(j, b)
        b2 = (b + 2) % 4

        @pl.when(j >= 2)
        def _():
            _sct(j - 2, b2).wait()

        @pl.when(j + 2 < NCHUNK)
        def _():
            _gat(j + 2, b2).start()

    scope = jax.named_scope("fused_edge_pass")
    scope.__enter__()
    _gat(0, 0).start()
    _gat(1, 1).start()

    def _ring(t, carry):
        j0 = 4 * t
        for b in range(4):
            _step(j0 + b, b)
        return carry
    lax.fori_loop(0, NQ, _ring, 0)
    _step(jnp.int32(NCHUNK - 2), 0)
    _step(jnp.int32(NCHUNK - 1), 1)
    _sct(NCHUNK - 2, 0).wait()
    _sct(NCHUNK - 1, 1).wait()

    def _dpush(q, carry):
        pltpu.async_copy(denloc.at[pl.ds(q * CHUNK, CHUNK)],
                         den_sh.at[idx_id.at[q]], semd, add=True)
        return carry
    lax.fori_loop(0, NP // CHUNK, _dpush, 0)

    def _dwait(q, carry):
        pltpu.make_async_copy(denloc.at[pl.ds(q * CHUNK, CHUNK)],
                              den_sh.at[idx_id.at[q]], semd).wait()
        return carry
    lax.fori_loop(0, NP // CHUNK, _dwait, 0)
    scope.__exit__(None, None, None)

    plsc.subcore_barrier()

    pltpu.sync_copy(den_sh.at[pl.ds(sid * RPT, RPT)], dbuf)
    one16 = jnp.ones((16,), jnp.float32)

    for k in range(RPT // CHUNK):
        base = sid * RPT + k * CHUNK
        pltpu.sync_copy(out_sh.at[pl.ds(base, CHUNK)], rb0)
        for g in range(CHUNK // 16):
            den = dbuf[pl.ds(k * CHUNK + g * 16, 16)]
            rden = one16 / (den + 1e-16)
            for i in range(16):
                a = rden[i]
                r = g * 16 + i
                for c in range(F // 32):
                    rb0[r, pl.ds(c * 16, 16)] = rb0[r, pl.ds(c * 16, 16)] * a
        pltpu.sync_copy(rb0, out_hbm.at[cid, pl.ds(base, CHUNK)])


def kernel(x, edge_index, batch, W1, a1s, a1d, b1, W2, a2s, a2d, b2,
           W3, a3s, a3d, b3, Wl, bl):
    src3 = edge_index[0].reshape(NS, NCHUNK, CHUNK)
    dst3 = edge_index[1].reshape(NS, NCHUNK, CHUNK)

    h1, asad1, m1 = _tc_prep1(x, W1, a1s, a1d)
    o1 = _sc_layer(h1, asad1, m1, src3, dst3)
    x1, h2, asad2, m2 = _tc_prep2(o1, b1, W2, a2s, a2d)
    o2 = _sc_layer(h2, asad2, m2, src3, dst3)
    x2, h3, asad3, m3 = _tc_prep2(o2, b2, W3, a3s, a3d)
    o3 = _sc_layer(h3, asad3, m3, src3, dst3)
    return _tc_final(x1, x2, o3, b3, batch, Wl, bl)


# speedup vs baseline: 1.0799x; 1.0009x over previous
import functools

import jax
import jax.numpy as jnp
from jax import lax
from jax.experimental import pallas as pl
from jax.experimental.pallas import tpu as pltpu
from jax.experimental.pallas import tpu_sc as plsc

N = 10000
E = 320000
D_IN = 128
F = 64
OUT = 10
G = 64

NC = 2
NS = 16
NP = 10240
RPT = NP // NS
EPT = E // NS
CHUNK = 80
NCHUNK = EPT // CHUNK
NQ = NCHUNK // 4


def _attention_rows(h, as_ref, ad_ref):
    a2 = jnp.concatenate([as_ref[...], ad_ref[...]], axis=0)
    asad = lax.dot_general(a2, h, (((1,), (1,)), ((), ())),
                           preferred_element_type=jnp.float32)
    m = jnp.maximum(jnp.max(asad[0:1, :]) + jnp.max(asad[1:2, :]), 0.0)
    return asad, m


def _tc_prep1_body(x_ref, w_ref, as_ref, ad_ref, h_ref, asad_ref, m_ref):
    h = jnp.dot(x_ref[...], w_ref[...], preferred_element_type=jnp.float32)
    h_ref[0] = h[:, :F // 2]
    h_ref[1] = h[:, F // 2:]
    asad, m = _attention_rows(h, as_ref, ad_ref)
    asad_ref[...] = asad
    m_ref[...] = jnp.full((8, 128), m, jnp.float32)


def _tc_prep2_body(o_ref, b_ref, w_ref, as_ref, ad_ref,
                   xl_ref, h_ref, asad_ref, m_ref):
    o = jnp.concatenate([o_ref[0, :N, :], o_ref[1, :N, :]], axis=1)
    xl = jnp.maximum(o + b_ref[...], 0.0)
    xl_ref[...] = xl
    h = jnp.dot(xl, w_ref[...], preferred_element_type=jnp.float32)
    h_ref[0] = h[:, :F // 2]
    h_ref[1] = h[:, F // 2:]
    asad, m = _attention_rows(h, as_ref, ad_ref)
    asad_ref[...] = asad
    m_ref[...] = jnp.full((8, 128), m, jnp.float32)


def _tc_final_body(x1_ref, x2_ref, o_ref, b3_ref, batch_ref,
                   wl_ref, bl_ref, out_ref):
    o = jnp.concatenate([o_ref[0, :N, :], o_ref[1, :N, :]], axis=1)
    x3 = jnp.maximum(o + b3_ref[...], 0.0)
    xc = jnp.concatenate([x1_ref[...], x2_ref[...], x3], axis=1)
    gid = lax.broadcasted_iota(jnp.int32, (G, N), 0)
    oh_t = (batch_ref[...] == gid).astype(jnp.float32)
    sums = lax.dot_general(oh_t, xc, (((1,), (0,)), ((), ())),
                           preferred_element_type=jnp.float32)
    ones = jnp.ones((N, 1), jnp.float32)
    counts = lax.dot_general(oh_t, ones, (((1,), (0,)), ((), ())),
                             preferred_element_type=jnp.float32)
    pooled = sums / jnp.maximum(counts, 1.0)
    logits = jnp.dot(pooled, wl_ref[...],
                     preferred_element_type=jnp.float32) + bl_ref[...]
    z = logits - jnp.max(logits, axis=1, keepdims=True)
    ez = jnp.exp(z)
    out_ref[...] = ez / jnp.sum(ez, axis=1, keepdims=True)


def _tc_prep1(x, w, a_s, a_d):
    return pl.pallas_call(
        _tc_prep1_body,
        out_shape=[
            jax.ShapeDtypeStruct((NC, N, F // 2), jnp.float32),
            jax.ShapeDtypeStruct((2, N), jnp.float32),
            jax.ShapeDtypeStruct((8, 128), jnp.float32),
        ],
    )(x, w, a_s.reshape(1, F), a_d.reshape(1, F))


def _tc_prep2(o, b, w, a_s, a_d):
    return pl.pallas_call(
        _tc_prep2_body,
        out_shape=[
            jax.ShapeDtypeStruct((N, F), jnp.float32),
            jax.ShapeDtypeStruct((NC, N, F // 2), jnp.float32),
            jax.ShapeDtypeStruct((2, N), jnp.float32),
            jax.ShapeDtypeStruct((8, 128), jnp.float32),
        ],
    )(o, b.reshape(1, F), w, a_s.reshape(1, F), a_d.reshape(1, F))


def _tc_final(x1, x2, o, b3, batch, wl, bl):
    return pl.pallas_call(
        _tc_final_body,
        out_shape=jax.ShapeDtypeStruct((G, OUT), jnp.float32),
    )(x1, x2, o, b3.reshape(1, F), batch.reshape(1, N), wl,
      bl.reshape(1, OUT))


_SC_MESH = plsc.VectorSubcoreMesh(core_axis_name="c", subcore_axis_name="s")


@functools.partial(
    pl.kernel,
    out_type=jax.ShapeDtypeStruct((NC, NP, F // 2), jnp.float32),
    mesh=_SC_MESH,
    compiler_params=pltpu.CompilerParams(
        needs_layout_passes=False, use_tc_tiling_on_sc=False),
    scratch_types=[
        pltpu.VMEM((NCHUNK, CHUNK), jnp.int32),
        pltpu.VMEM((NCHUNK, CHUNK), jnp.int32),
        pltpu.VMEM((NP,), jnp.float32),
        pltpu.VMEM((NP // CHUNK, CHUNK), jnp.int32),
        pltpu.VMEM((N,), jnp.float32),
        pltpu.VMEM((N,), jnp.float32),
        pltpu.VMEM((CHUNK, F // 2), jnp.float32),
        pltpu.VMEM((CHUNK, F // 2), jnp.float32),
        pltpu.VMEM((CHUNK, F // 2), jnp.float32),
        pltpu.VMEM((CHUNK, F // 2), jnp.float32),
        pltpu.VMEM((RPT,), jnp.float32),
        pltpu.VMEM((16,), jnp.float32),
        pltpu.VMEM_SHARED((NP,), jnp.float32),
        pltpu.VMEM_SHARED((NP, F // 2), jnp.float32),
        pltpu.SemaphoreType.DMA,
        pltpu.SemaphoreType.DMA,
        pltpu.SemaphoreType.DMA,
        pltpu.SemaphoreType.DMA,
        pltpu.SemaphoreType.DMA,
        pltpu.SemaphoreType.DMA,
        pltpu.SemaphoreType.DMA,
        pltpu.SemaphoreType.DMA,
        pltpu.SemaphoreType.DMA,
    ],
)
def _sc_layer(h_hbm, asad_hbm, m_hbm, src_hbm, dst_hbm, out_hbm,
              src_v, dst_v, denloc, idx_id, as_v, ad_v, rb0, rb1, rb2, rb3,
              dbuf, m_v, den_sh, out_sh, semd, g0, g1, g2, g3, s0, s1, s2, s3):
    sid = lax.axis_index("s")
    cid = lax.axis_index("c")
    bufs = (rb0, rb1, rb2, rb3)
    gsems = (g0, g1, g2, g3)
    ssems = (s0, s1, s2, s3)

    pltpu.sync_copy(src_hbm.at[sid], src_v)
    pltpu.sync_copy(dst_hbm.at[sid], dst_v)
    pltpu.sync_copy(asad_hbm.at[0], as_v)
    pltpu.sync_copy(asad_hbm.at[1], ad_v)
    pltpu.sync_copy(m_hbm.at[0, pl.ds(0, 16)], m_v)

    zero16 = jnp.zeros((16,), jnp.float32)
    zero16i = jnp.zeros((16,), jnp.int32)

    def _zrow(r, carry):
        for c in range(F // 32):
            rb0[r, pl.ds(c * 16, 16)] = zero16
        return carry
    lax.fori_loop(0, CHUNK, _zrow, 0)
    for k in range(RPT // CHUNK):
        pltpu.sync_copy(rb0, out_sh.at[pl.ds(sid * RPT + k * CHUNK, CHUNK)])

    def _zden(r, carry):
        dbuf[pl.ds(r * 16, 16)] = zero16
        return carry
    lax.fori_loop(0, RPT // 16, _zden, 0)
    pltpu.sync_copy(dbuf, den_sh.at[pl.ds(sid * RPT, RPT)])

    iota16 = lax.iota(jnp.int32, 16)

    def _zdl(q, carry):
        for g in range(CHUNK // 16):
            denloc[pl.ds(q * CHUNK + g * 16, 16)] = zero16
            idx_id[q, pl.ds(g * 16, 16)] = iota16 + (q * CHUNK + g * 16)
        return carry
    lax.fori_loop(0, NP // CHUNK, _zdl, 0)

    plsc.subcore_barrier()

    m_vec = m_v[...]
    h_half = h_hbm.at[cid]

    def _gat(j, b):
        return pltpu.make_async_copy(h_half.at[src_v.at[j]], bufs[b], gsems[b])

    def _sct(j, b):
        return pltpu.make_async_copy(bufs[b], out_sh.at[dst_v.at[j]], ssems[b])

    def _proc(j, b):
        buf = bufs[b]
        for g in range(CHUNK // 16):
            sv = src_v[j, pl.ds(g * 16, 16)]
            dv = dst_v[j, pl.ds(g * 16, 16)]
            e = plsc.load_gather(as_v, [sv]) + plsc.load_gather(ad_v, [dv])
            e = jnp.where(e >= 0.0, e, e * 0.2)
            w = jnp.exp(e - m_vec)
            plsc.addupdate_scatter(denloc, [dv], w)
            for i in range(16):
                a = w[i]
                r = g * 16 + i
                for c in range(F // 32):
                    buf[r, pl.ds(c * 16, 16)] = buf[r, pl.ds(c * 16, 16)] * a
        _sct(j, b).start(add=True)

    def _step(j, b):
        _gat(j, b).wait()
        _proc(j, b)
        b2 = (b + 2) % 4

        @pl.when(j >= 2)
        def _():
            _sct(j - 2, b2).wait()

        @pl.when(j + 2 < NCHUNK)
        def _():
            _gat(j + 2, b2).start()

    scope = jax.named_scope("fused_edge_pass")
    scope.__enter__()
    _gat(0, 0).start()
    _gat(1, 1).start()

    def _ring(t, carry):
        j0 = 4 * t
        for b in range(4):
            _step(j0 + b, b)
        return carry
    lax.fori_loop(0, NQ, _ring, 0)
    _step(jnp.int32(NCHUNK - 2), 0)
    _step(jnp.int32(NCHUNK - 1), 1)
    _sct(NCHUNK - 2, 0).wait()
    _sct(NCHUNK - 1, 1).wait()

    def _dpush(q, carry):
        pltpu.async_copy(denloc.at[pl.ds(q * CHUNK, CHUNK)],
                         den_sh.at[idx_id.at[q]], semd, add=True)
        return carry
    lax.fori_loop(0, NP // CHUNK, _dpush, 0)

    def _dwait(q, carry):
        pltpu.make_async_copy(denloc.at[pl.ds(q * CHUNK, CHUNK)],
                              den_sh.at[idx_id.at[q]], semd).wait()
        return carry
    lax.fori_loop(0, NP // CHUNK, _dwait, 0)
    scope.__exit__(None, None, None)

    plsc.subcore_barrier()

    pltpu.sync_copy(den_sh.at[pl.ds(sid * RPT, RPT)], dbuf)
    one16 = jnp.ones((16,), jnp.float32)

    for k in range(RPT // CHUNK):
        base = sid * RPT + k * CHUNK
        pltpu.sync_copy(out_sh.at[pl.ds(base, CHUNK)], rb0)
        for g in range(CHUNK // 16):
            den = dbuf[pl.ds(k * CHUNK + g * 16, 16)]
            rden = one16 / (den + 1e-16)
            for i in range(16):
                a = rden[i]
                r = g * 16 + i
                for c in range(F // 32):
                    rb0[r, pl.ds(c * 16, 16)] = rb0[r, pl.ds(c * 16, 16)] * a
        pltpu.sync_copy(rb0, out_hbm.at[cid, pl.ds(base, CHUNK)])


def kernel(x, edge_index, batch, W1, a1s, a1d, b1, W2, a2s, a2d, b2,
           W3, a3s, a3d, b3, Wl, bl):
    src3 = edge_index[0].reshape(NS, NCHUNK, CHUNK)
    dst3 = edge_index[1].reshape(NS, NCHUNK, CHUNK)

    h1, asad1, m1 = _tc_prep1(x, W1, a1s, a1d)
    o1 = _sc_layer(h1, asad1, m1, src3, dst3)
    x1, h2, asad2, m2 = _tc_prep2(o1, b1, W2, a2s, a2d)
    o2 = _sc_layer(h2, asad2, m2, src3, dst3)
    x2, h3, asad3, m3 = _tc_prep2(o2, b2, W3, a3s, a3d)
    o3 = _sc_layer(h3, asad3, m3, src3, dst3)
    return _tc_final(x1, x2, o3, b3, batch, Wl, bl)


# speedup vs baseline: 1.0802x; 1.0002x over previous
import functools

import jax
import jax.numpy as jnp
from jax import lax
from jax.experimental import pallas as pl
from jax.experimental.pallas import tpu as pltpu
from jax.experimental.pallas import tpu_sc as plsc

N = 10000
E = 320000
D_IN = 128
F = 64
OUT = 10
G = 64

NC = 2
NS = 16
NP = 10240
RPT = NP // NS
EPT = E // NS
CHUNK = 80
NCHUNK = EPT // CHUNK
NQ = NCHUNK // 4


def _attention_rows(h, as_ref, ad_ref):
    a2 = jnp.concatenate([as_ref[...], ad_ref[...]], axis=0)
    asad = lax.dot_general(a2, h, (((1,), (1,)), ((), ())),
                           preferred_element_type=jnp.float32)
    m = jnp.maximum(jnp.max(asad[0:1, :]) + jnp.max(asad[1:2, :]), 0.0)
    return asad, m


def _tc_prep1_body(x_ref, w_ref, as_ref, ad_ref, h_ref, asad_ref, m_ref):
    h = jnp.dot(x_ref[...], w_ref[...], preferred_element_type=jnp.float32)
    h_ref[0] = h[:, :F // 2]
    h_ref[1] = h[:, F // 2:]
    asad, m = _attention_rows(h, as_ref, ad_ref)
    asad_ref[...] = asad
    m_ref[...] = jnp.full((8, 128), m, jnp.float32)


def _tc_prep2_body(o_ref, b_ref, w_ref, as_ref, ad_ref,
                   xl_ref, h_ref, asad_ref, m_ref):
    o = jnp.concatenate([o_ref[0, :N, :], o_ref[1, :N, :]], axis=1)
    xl = jnp.maximum(o + b_ref[...], 0.0)
    xl_ref[...] = xl
    h = jnp.dot(xl, w_ref[...], preferred_element_type=jnp.float32)
    h_ref[0] = h[:, :F // 2]
    h_ref[1] = h[:, F // 2:]
    asad, m = _attention_rows(h, as_ref, ad_ref)
    asad_ref[...] = asad
    m_ref[...] = jnp.full((8, 128), m, jnp.float32)


def _tc_final_body(x1_ref, x2_ref, o_ref, b3_ref, batch_ref,
                   wl_ref, bl_ref, out_ref):
    o = jnp.concatenate([o_ref[0, :N, :], o_ref[1, :N, :]], axis=1)
    x3 = jnp.maximum(o + b3_ref[...], 0.0)
    xc = jnp.concatenate([x1_ref[...], x2_ref[...], x3], axis=1)
    gid = lax.broadcasted_iota(jnp.int32, (G, N), 0)
    oh_t = (batch_ref[...] == gid).astype(jnp.float32)
    sums = lax.dot_general(oh_t, xc, (((1,), (0,)), ((), ())),
                           preferred_element_type=jnp.float32)
    ones = jnp.ones((N, 1), jnp.float32)
    counts = lax.dot_general(oh_t, ones, (((1,), (0,)), ((), ())),
                             preferred_element_type=jnp.float32)
    pooled = sums / jnp.maximum(counts, 1.0)
    logits = jnp.dot(pooled, wl_ref[...],
                     preferred_element_type=jnp.float32) + bl_ref[...]
    z = logits - jnp.max(logits, axis=1, keepdims=True)
    ez = jnp.exp(z)
    out_ref[...] = ez / jnp.sum(ez, axis=1, keepdims=True)


def _tc_prep1(x, w, a_s, a_d):
    return pl.pallas_call(
        _tc_prep1_body,
        out_shape=[
            jax.ShapeDtypeStruct((NC, N, F // 2), jnp.float32),
            jax.ShapeDtypeStruct((2, N), jnp.float32),
            jax.ShapeDtypeStruct((8, 128), jnp.float32),
        ],
    )(x, w, a_s.reshape(1, F), a_d.reshape(1, F))


def _tc_prep2(o, b, w, a_s, a_d):
    return pl.pallas_call(
        _tc_prep2_body,
        out_shape=[
            jax.ShapeDtypeStruct((N, F), jnp.float32),
            jax.ShapeDtypeStruct((NC, N, F // 2), jnp.float32),
            jax.ShapeDtypeStruct((2, N), jnp.float32),
            jax.ShapeDtypeStruct((8, 128), jnp.float32),
        ],
    )(o, b.reshape(1, F), w, a_s.reshape(1, F), a_d.reshape(1, F))


def _tc_final(x1, x2, o, b3, batch, wl, bl):
    return pl.pallas_call(
        _tc_final_body,
        out_shape=jax.ShapeDtypeStruct((G, OUT), jnp.float32),
    )(x1, x2, o, b3.reshape(1, F), batch.reshape(1, N), wl,
      bl.reshape(1, OUT))


_SC_MESH = plsc.VectorSubcoreMesh(core_axis_name="c", subcore_axis_name="s")


@functools.partial(
    pl.kernel,
    out_type=jax.ShapeDtypeStruct((NC, NP, F // 2), jnp.float32),
    mesh=_SC_MESH,
    compiler_params=pltpu.CompilerParams(
        needs_layout_passes=False, use_tc_tiling_on_sc=False),
    scratch_types=[
        pltpu.VMEM((NCHUNK, CHUNK), jnp.int32),
        pltpu.VMEM((NCHUNK, CHUNK), jnp.int32),
        pltpu.VMEM((NP,), jnp.float32),
        pltpu.VMEM((NP // CHUNK, CHUNK), jnp.int32),
        pltpu.VMEM((N,), jnp.float32),
        pltpu.VMEM((N,), jnp.float32),
        pltpu.VMEM((CHUNK, F // 2), jnp.float32),
        pltpu.VMEM((CHUNK, F // 2), jnp.float32),
        pltpu.VMEM((CHUNK, F // 2), jnp.float32),
        pltpu.VMEM((CHUNK, F // 2), jnp.float32),
        pltpu.VMEM((RPT,), jnp.float32),
        pltpu.VMEM((16,), jnp.float32),
        pltpu.VMEM_SHARED((NP,), jnp.float32),
        pltpu.VMEM_SHARED((NP, F // 2), jnp.float32),
        pltpu.SemaphoreType.DMA,
        pltpu.SemaphoreType.DMA,
        pltpu.SemaphoreType.DMA,
        pltpu.SemaphoreType.DMA,
        pltpu.SemaphoreType.DMA,
        pltpu.SemaphoreType.DMA,
        pltpu.SemaphoreType.DMA,
        pltpu.SemaphoreType.DMA,
        pltpu.SemaphoreType.DMA,
    ],
)
def _sc_layer(h_hbm, asad_hbm, m_hbm, src_hbm, dst_hbm, out_hbm,
              src_v, dst_v, denloc, idx_id, as_v, ad_v, rb0, rb1, rb2, rb3,
              dbuf, m_v, den_sh, out_sh, semd, g0, g1, g2, g3, s0, s1, s2, s3):
    sid = lax.axis_index("s")
    cid = lax.axis_index("c")
    bufs = (rb0, rb1, rb2, rb3)
    gsems = (g0, g1, g2, g3)
    ssems = (s0, s1, s2, s3)

    pltpu.sync_copy(src_hbm.at[sid], src_v)
    pltpu.sync_copy(dst_hbm.at[sid], dst_v)
    pltpu.sync_copy(asad_hbm.at[0], as_v)
    pltpu.sync_copy(asad_hbm.at[1], ad_v)
    pltpu.sync_copy(m_hbm.at[0, pl.ds(0, 16)], m_v)

    zero16 = jnp.zeros((16,), jnp.float32)

    def _zrow(r, carry):
        for c in range(F // 32):
            rb0[r, pl.ds(c * 16, 16)] = zero16
        return carry
    lax.fori_loop(0, CHUNK, _zrow, 0)
    for k in range(RPT // CHUNK):
        pltpu.sync_copy(rb0, out_sh.at[pl.ds(sid * RPT + k * CHUNK, CHUNK)])

    def _zden(r, carry):
        dbuf[pl.ds(r * 16, 16)] = zero16
        return carry
    lax.fori_loop(0, RPT // 16, _zden, 0)
    pltpu.sync_copy(dbuf, den_sh.at[pl.ds(sid * RPT, RPT)])

    iota16 = lax.iota(jnp.int32, 16)

    def _zdl(q, carry):
        for g in range(CHUNK // 16):
            denloc[pl.ds(q * CHUNK + g * 16, 16)] = zero16
            idx_id[q, pl.ds(g * 16, 16)] = iota16 + (q * CHUNK + g * 16)
        return carry
    lax.fori_loop(0, NP // CHUNK, _zdl, 0)

    plsc.subcore_barrier()

    m_vec = m_v[...]
    h_half = h_hbm.at[cid]

    def _gat(j, b):
        return pltpu.make_async_copy(h_half.at[src_v.at[j]], bufs[b], gsems[b])

    def _sct(j, b):
        return pltpu.make_async_copy(bufs[b], out_sh.at[dst_v.at[j]], ssems[b])

    def _proc(j, b):
        buf = bufs[b]
        for g in range(CHUNK // 16):
            sv = src_v[j, pl.ds(g * 16, 16)]
            dv = dst_v[j, pl.ds(g * 16, 16)]
            e = plsc.load_gather(as_v, [sv]) + plsc.load_gather(ad_v, [dv])
            e = jnp.where(e >= 0.0, e, e * 0.2)
            w = jnp.exp(e - m_vec)
            plsc.addupdate_scatter(denloc, [dv], w)
            for i in range(16):
                a = w[i]
                r = g * 16 + i
                for c in range(F // 32):
                    buf[r, pl.ds(c * 16, 16)] = buf[r, pl.ds(c * 16, 16)] * a
        _sct(j, b).start(add=True)

    def _step(j, b):
        _gat(j, b).wait()
        _proc(j, b)
        b2 = (b + 2) % 4

        @pl.when(j >= 2)
        def _():
            _sct(j - 2, b2).wait()

        @pl.when(j + 2 < NCHUNK)
        def _():
            _gat(j + 2, b2).start()

    scope = jax.named_scope("fused_edge_pass")
    scope.__enter__()
    _gat(0, 0).start()
    _gat(1, 1).start()

    def _ring(t, carry):
        j0 = 4 * t
        for b in range(4):
            _step(j0 + b, b)
        return carry
    lax.fori_loop(0, NQ, _ring, 0)
    _step(jnp.int32(NCHUNK - 2), 0)
    _step(jnp.int32(NCHUNK - 1), 1)
    _sct(NCHUNK - 2, 0).wait()
    _sct(NCHUNK - 1, 1).wait()

    def _dpush(q, carry):
        pltpu.async_copy(denloc.at[pl.ds(q * CHUNK, CHUNK)],
                         den_sh.at[idx_id.at[q]], semd, add=True)
        return carry
    lax.fori_loop(0, NP // CHUNK, _dpush, 0)

    def _dwait(q, carry):
        pltpu.make_async_copy(denloc.at[pl.ds(q * CHUNK, CHUNK)],
                              den_sh.at[idx_id.at[q]], semd).wait()
        return carry
    lax.fori_loop(0, NP // CHUNK, _dwait, 0)
    scope.__exit__(None, None, None)

    plsc.subcore_barrier()

    pltpu.sync_copy(den_sh.at[pl.ds(sid * RPT, RPT)], dbuf)
    one16 = jnp.ones((16,), jnp.float32)

    for k in range(RPT // CHUNK):
        base = sid * RPT + k * CHUNK
        pltpu.sync_copy(out_sh.at[pl.ds(base, CHUNK)], rb0)
        for g in range(CHUNK // 16):
            den = dbuf[pl.ds(k * CHUNK + g * 16, 16)]
            rden = one16 / (den + 1e-16)
            for i in range(16):
                a = rden[i]
                r = g * 16 + i
                for c in range(F // 32):
                    rb0[r, pl.ds(c * 16, 16)] = rb0[r, pl.ds(c * 16, 16)] * a
        pltpu.sync_copy(rb0, out_hbm.at[cid, pl.ds(base, CHUNK)])


def kernel(x, edge_index, batch, W1, a1s, a1d, b1, W2, a2s, a2d, b2,
           W3, a3s, a3d, b3, Wl, bl):
    src3 = edge_index[0].reshape(NS, NCHUNK, CHUNK)
    dst3 = edge_index[1].reshape(NS, NCHUNK, CHUNK)

    h1, asad1, m1 = _tc_prep1(x, W1, a1s, a1d)
    o1 = _sc_layer(h1, asad1, m1, src3, dst3)
    x1, h2, asad2, m2 = _tc_prep2(o1, b1, W2, a2s, a2d)
    o2 = _sc_layer(h2, asad2, m2, src3, dst3)
    x2, h3, asad3, m3 = _tc_prep2(o2, b2, W3, a3s, a3d)
    o3 = _sc_layer(h3, asad3, m3, src3, dst3)
    return _tc_final(x1, x2, o3, b3, batch, Wl, bl)


# speedup vs baseline: 1.1032x; 1.0213x over previous
import functools

import jax
import jax.numpy as jnp
from jax import lax
from jax.experimental import pallas as pl
from jax.experimental.pallas import tpu as pltpu
from jax.experimental.pallas import tpu_sc as plsc

N = 10000
E = 320000
D_IN = 128
F = 64
OUT = 10
G = 64

NC = 2
NS = 16
NP = 10240
RPT = NP // NS
EPT = E // NS
CHUNK = 80
NCHUNK = EPT // CHUNK
NQ = NCHUNK // 4


def _attention_rows(h, as_ref, ad_ref):
    a2 = jnp.concatenate([as_ref[...], ad_ref[...]], axis=0)
    asad = lax.dot_general(a2, h, (((1,), (1,)), ((), ())),
                           preferred_element_type=jnp.float32)
    m = jnp.maximum(jnp.max(asad[0:1, :]) + jnp.max(asad[1:2, :]), 0.0)
    return asad, m


def _tc_prep1_body(x_ref, w_ref, as_ref, ad_ref, h_ref, asad_ref, m_ref):
    h = jnp.dot(x_ref[...], w_ref[...], preferred_element_type=jnp.float32)
    h_ref[0] = h[:, :F // 2]
    h_ref[1] = h[:, F // 2:]
    asad, m = _attention_rows(h, as_ref, ad_ref)
    asad_ref[...] = asad
    m_ref[...] = jnp.full((8, 128), m, jnp.float32)


def _tc_prep2_body(o_ref, b_ref, w_ref, as_ref, ad_ref,
                   xl_ref, h_ref, asad_ref, m_ref):
    o = jnp.concatenate([o_ref[0, :N, :], o_ref[1, :N, :]], axis=1)
    xl = jnp.maximum(o + b_ref[...], 0.0)
    xl_ref[...] = xl
    h = jnp.dot(xl, w_ref[...], preferred_element_type=jnp.float32)
    h_ref[0] = h[:, :F // 2]
    h_ref[1] = h[:, F // 2:]
    asad, m = _attention_rows(h, as_ref, ad_ref)
    asad_ref[...] = asad
    m_ref[...] = jnp.full((8, 128), m, jnp.float32)


def _tc_final_body(x1_ref, x2_ref, o_ref, b3_ref, batch_ref,
                   wl_ref, bl_ref, out_ref):
    o = jnp.concatenate([o_ref[0, :N, :], o_ref[1, :N, :]], axis=1)
    x3 = jnp.maximum(o + b3_ref[...], 0.0)
    xc = jnp.concatenate([x1_ref[...], x2_ref[...], x3], axis=1)
    gid = lax.broadcasted_iota(jnp.int32, (G, N), 0)
    oh_t = (batch_ref[...] == gid).astype(jnp.float32)
    sums = lax.dot_general(oh_t, xc, (((1,), (0,)), ((), ())),
                           preferred_element_type=jnp.float32)
    ones = jnp.ones((N, 1), jnp.float32)
    counts = lax.dot_general(oh_t, ones, (((1,), (0,)), ((), ())),
                             preferred_element_type=jnp.float32)
    pooled = sums / jnp.maximum(counts, 1.0)
    logits = jnp.dot(pooled, wl_ref[...],
                     preferred_element_type=jnp.float32) + bl_ref[...]
    z = logits - jnp.max(logits, axis=1, keepdims=True)
    ez = jnp.exp(z)
    out_ref[...] = ez / jnp.sum(ez, axis=1, keepdims=True)


def _tc_prep1(x, w, a_s, a_d):
    return pl.pallas_call(
        _tc_prep1_body,
        out_shape=[
            jax.ShapeDtypeStruct((NC, N, F // 2), jnp.float32),
            jax.ShapeDtypeStruct((2, N), jnp.float32),
            jax.ShapeDtypeStruct((8, 128), jnp.float32),
        ],
    )(x, w, a_s.reshape(1, F), a_d.reshape(1, F))


def _tc_prep2(o, b, w, a_s, a_d):
    return pl.pallas_call(
        _tc_prep2_body,
        out_shape=[
            jax.ShapeDtypeStruct((N, F), jnp.float32),
            jax.ShapeDtypeStruct((NC, N, F // 2), jnp.float32),
            jax.ShapeDtypeStruct((2, N), jnp.float32),
            jax.ShapeDtypeStruct((8, 128), jnp.float32),
        ],
    )(o, b.reshape(1, F), w, a_s.reshape(1, F), a_d.reshape(1, F))


def _tc_final(x1, x2, o, b3, batch, wl, bl):
    return pl.pallas_call(
        _tc_final_body,
        out_shape=jax.ShapeDtypeStruct((G, OUT), jnp.float32),
    )(x1, x2, o, b3.reshape(1, F), batch.reshape(1, N), wl,
      bl.reshape(1, OUT))


_SC_MESH = plsc.VectorSubcoreMesh(core_axis_name="c", subcore_axis_name="s")


@functools.partial(
    pl.kernel,
    out_type=jax.ShapeDtypeStruct((NC, NP, F // 2), jnp.float32),
    mesh=_SC_MESH,
    compiler_params=pltpu.CompilerParams(
        needs_layout_passes=False, use_tc_tiling_on_sc=False),
    scratch_types=[
        pltpu.VMEM((NCHUNK, CHUNK), jnp.int32),
        pltpu.VMEM((NCHUNK, CHUNK), jnp.int32),
        pltpu.VMEM((NP,), jnp.float32),
        pltpu.VMEM((NP // CHUNK, CHUNK), jnp.int32),
        pltpu.VMEM((N,), jnp.float32),
        pltpu.VMEM((N,), jnp.float32),
        pltpu.VMEM((CHUNK, F // 2), jnp.float32),
        pltpu.VMEM((CHUNK, F // 2), jnp.float32),
        pltpu.VMEM((CHUNK, F // 2), jnp.float32),
        pltpu.VMEM((CHUNK, F // 2), jnp.float32),
        pltpu.VMEM((CHUNK, F // 2), jnp.float32),
        pltpu.VMEM((RPT,), jnp.float32),
        pltpu.VMEM((16,), jnp.float32),
        pltpu.VMEM_SHARED((NP,), jnp.float32),
        pltpu.VMEM_SHARED((NP, F // 2), jnp.float32),
        pltpu.SemaphoreType.DMA,
        pltpu.SemaphoreType.DMA,
        pltpu.SemaphoreType.DMA,
        pltpu.SemaphoreType.DMA,
        pltpu.SemaphoreType.DMA,
        pltpu.SemaphoreType.DMA,
        pltpu.SemaphoreType.DMA,
        pltpu.SemaphoreType.DMA,
        pltpu.SemaphoreType.DMA,
    ],
)
def _sc_layer(h_hbm, asad_hbm, m_hbm, src_hbm, dst_hbm, out_hbm,
              src_v, dst_v, denloc, idx_id, as_v, ad_v, rb0, rb1, rb2, rb3,
              zbuf, dbuf, m_v, den_sh, out_sh,
              semd, g0, g1, g2, g3, s0, s1, s2, s3):
    sid = lax.axis_index("s")
    cid = lax.axis_index("c")
    bufs = (rb0, rb1, rb2, rb3)
    gsems = (g0, g1, g2, g3)
    ssems = (s0, s1, s2, s3)
    h_half = h_hbm.at[cid]

    def _gat(j, b):
        return pltpu.make_async_copy(h_half.at[src_v.at[j]], bufs[b], gsems[b])

    pltpu.sync_copy(src_hbm.at[sid], src_v)
    cp_dst = pltpu.make_async_copy(dst_hbm.at[sid], dst_v, semd)
    cp_as = pltpu.make_async_copy(asad_hbm.at[0], as_v, semd)
    cp_ad = pltpu.make_async_copy(asad_hbm.at[1], ad_v, semd)
    cp_m = pltpu.make_async_copy(m_hbm.at[0, pl.ds(0, 16)], m_v, semd)
    cp_dst.start()
    cp_as.start()
    cp_ad.start()
    cp_m.start()
    _gat(0, 0).start()
    _gat(1, 1).start()

    zero16 = jnp.zeros((16,), jnp.float32)

    def _zrow(r, carry):
        for c in range(F // 32):
            zbuf[r, pl.ds(c * 16, 16)] = zero16
        return carry
    lax.fori_loop(0, CHUNK, _zrow, 0)
    for k in range(RPT // CHUNK):
        pltpu.sync_copy(zbuf, out_sh.at[pl.ds(sid * RPT + k * CHUNK, CHUNK)])

    def _zden(r, carry):
        dbuf[pl.ds(r * 16, 16)] = zero16
        return carry
    lax.fori_loop(0, RPT // 16, _zden, 0)
    pltpu.sync_copy(dbuf, den_sh.at[pl.ds(sid * RPT, RPT)])

    iota16 = lax.iota(jnp.int32, 16)

    def _zdl(q, carry):
        for g in range(CHUNK // 16):
            denloc[pl.ds(q * CHUNK + g * 16, 16)] = zero16
            idx_id[q, pl.ds(g * 16, 16)] = iota16 + (q * CHUNK + g * 16)
        return carry
    lax.fori_loop(0, NP // CHUNK, _zdl, 0)

    cp_dst.wait()
    cp_as.wait()
    cp_ad.wait()
    cp_m.wait()

    plsc.subcore_barrier()

    m_vec = m_v[...]

    def _sct(j, b):
        return pltpu.make_async_copy(bufs[b], out_sh.at[dst_v.at[j]], ssems[b])

    def _proc(j, b):
        buf = bufs[b]
        for g in range(CHUNK // 16):
            sv = src_v[j, pl.ds(g * 16, 16)]
            dv = dst_v[j, pl.ds(g * 16, 16)]
            e = plsc.load_gather(as_v, [sv]) + plsc.load_gather(ad_v, [dv])
            e = jnp.where(e >= 0.0, e, e * 0.2)
            w = jnp.exp(e - m_vec)
            plsc.addupdate_scatter(denloc, [dv], w)
            for i in range(16):
                a = w[i]
                r = g * 16 + i
                for c in range(F // 32):
                    buf[r, pl.ds(c * 16, 16)] = buf[r, pl.ds(c * 16, 16)] * a
        _sct(j, b).start(add=True)

    def _step(j, b):
        _gat(j, b).wait()
        _proc(j, b)
        b2 = (b + 2) % 4

        @pl.when(j >= 2)
        def _():
            _sct(j - 2, b2).wait()

        @pl.when(j + 2 < NCHUNK)
        def _():
            _gat(j + 2, b2).start()

    scope = jax.named_scope("fused_edge_pass")
    scope.__enter__()

    def _ring(t, carry):
        j0 = 4 * t
        for b in range(4):
            _step(j0 + b, b)
        return carry
    lax.fori_loop(0, NQ, _ring, 0)
    _step(jnp.int32(NCHUNK - 2), 0)
    _step(jnp.int32(NCHUNK - 1), 1)
    _sct(NCHUNK - 2, 0).wait()
    _sct(NCHUNK - 1, 1).wait()

    def _dpush(q, carry):
        pltpu.async_copy(denloc.at[pl.ds(q * CHUNK, CHUNK)],
                         den_sh.at[idx_id.at[q]], semd, add=True)
        return carry
    lax.fori_loop(0, NP // CHUNK, _dpush, 0)

    def _dwait(q, carry):
        pltpu.make_async_copy(denloc.at[pl.ds(q * CHUNK, CHUNK)],
                              den_sh.at[idx_id.at[q]], semd).wait()
        return carry
    lax.fori_loop(0, NP // CHUNK, _dwait, 0)
    scope.__exit__(None, None, None)

    plsc.subcore_barrier()

    pltpu.sync_copy(den_sh.at[pl.ds(sid * RPT, RPT)], dbuf)
    one16 = jnp.ones((16,), jnp.float32)

    for k in range(RPT // CHUNK):
        base = sid * RPT + k * CHUNK
        pltpu.sync_copy(out_sh.at[pl.ds(base, CHUNK)], rb0)
        for g in range(CHUNK // 16):
            den = dbuf[pl.ds(k * CHUNK + g * 16, 16)]
            rden = one16 / (den + 1e-16)
            for i in range(16):
                a = rden[i]
                r = g * 16 + i
                for c in range(F // 32):
                    rb0[r, pl.ds(c * 16, 16)] = rb0[r, pl.ds(c * 16, 16)] * a
        pltpu.sync_copy(rb0, out_hbm.at[cid, pl.ds(base, CHUNK)])


def kernel(x, edge_index, batch, W1, a1s, a1d, b1, W2, a2s, a2d, b2,
           W3, a3s, a3d, b3, Wl, bl):
    src3 = edge_index[0].reshape(NS, NCHUNK, CHUNK)
    dst3 = edge_index[1].reshape(NS, NCHUNK, CHUNK)

    h1, asad1, m1 = _tc_prep1(x, W1, a1s, a1d)
    o1 = _sc_layer(h1, asad1, m1, src3, dst3)
    x1, h2, asad2, m2 = _tc_prep2(o1, b1, W2, a2s, a2d)
    o2 = _sc_layer(h2, asad2, m2, src3, dst3)
    x2, h3, asad3, m3 = _tc_prep2(o2, b2, W3, a3s, a3d)
    o3 = _sc_layer(h3, asad3, m3, src3, dst3)
    return _tc_final(x1, x2, o3, b3, batch, Wl, bl)
